# baseline (speedup 1.0000x reference)
import functools

import jax
import jax.numpy as jnp
from jax import lax
from jax.experimental import pallas as pl
from jax.experimental.pallas import tpu as pltpu

_BN_EPS = 1e-5


def _upconv_stats_kernel(x_ref, r_ref, w_ref, mask_ref, y_ref, sum_ref, ssq_ref,
                         *, wp, lout):
    xb = x_ref[...].astype(jnp.bfloat16)
    xr = jnp.dot(xb, r_ref[...], preferred_element_type=jnp.float32)
    xrb = xr.astype(jnp.bfloat16)
    cout = w_ref.shape[1]
    acc = jnp.zeros((cout, lout), jnp.float32)
    for t in range(9):
        s = (t // 3) * wp + (t % 3)
        acc = acc + jnp.dot(w_ref[t], xrb[:, s:s + lout],
                            preferred_element_type=jnp.float32)
    y_ref[...] = acc.astype(jnp.bfloat16)
    yv = acc * mask_ref[...]
    sum_ref[...] = jnp.sum(yv, axis=1, keepdims=True)
    ssq_ref[...] = jnp.sum(yv * yv, axis=1, keepdims=True)


def _bn_relu_slice_kernel(y_ref, a_ref, b_ref, o_ref, *, wo):
    cout = y_ref.shape[0]
    y = y_ref[:, :, :wo].astype(jnp.float32)
    a = jnp.reshape(a_ref[...], (cout, 1, 1))
    b = jnp.reshape(b_ref[...], (cout, 1, 1))
    o_ref[...] = jnp.maximum(y * a + b, 0.0)


def _interp_matrix(in_size, out_size):
    if out_size == 1:
        src = jnp.zeros((1,), jnp.float32)
    else:
        src = jnp.arange(out_size, dtype=jnp.float32) * ((in_size - 1) / (out_size - 1))
    i0 = jnp.clip(jnp.floor(src).astype(jnp.int32), 0, in_size - 1)
    i1 = jnp.clip(i0 + 1, 0, in_size - 1)
    frac = src - i0.astype(jnp.float32)
    rows = jnp.arange(out_size)
    m = jnp.zeros((out_size, in_size), jnp.float32)
    m = m.at[rows, i0].add(1.0 - frac)
    m = m.at[rows, i1].add(frac)
    return m


@functools.partial(jax.jit, static_argnames=("out_hw",))
def _up_conv_impl(x, conv_w, bn_gamma, bn_beta, *, out_hw):
    n, cin, h, w = x.shape
    ho, wo = out_hw
    cout = conv_w.shape[0]
    hw = h * w
    wp = wo + 2
    hpp = ho + 3
    lpad = hpp * wp
    lout = ho * wp

    rh = jnp.zeros((hpp, h), jnp.float32).at[1:ho + 1].set(_interp_matrix(h, ho))
    rw = jnp.zeros((w, wp), jnp.float32).at[:, 1:wo + 1].set(_interp_matrix(w, wo).T)
    r_up = jnp.einsum('ih,wj->hwij', rh, rw).reshape(hw, lpad).astype(jnp.bfloat16)

    w_taps = conv_w.astype(jnp.bfloat16).transpose(2, 3, 0, 1).reshape(9, cout, cin)
    mask = ((jnp.arange(lout) % wp) < wo).astype(jnp.float32).reshape(1, lout)
    x_flat = x.reshape(n, cin, hw)

    vmem_limit = 64 * 1024 * 1024

    kern_a = functools.partial(_upconv_stats_kernel, wp=wp, lout=lout)
    y_conv, ch_sum, ch_ssq = pl.pallas_call(
        kern_a,
        grid=(n,),
        in_specs=[
            pl.BlockSpec((None, cin, hw), lambda i: (i, 0, 0)),
            pl.BlockSpec((hw, lpad), lambda i: (0, 0)),
            pl.BlockSpec((9, cout, cin), lambda i: (0, 0, 0)),
            pl.BlockSpec((1, lout), lambda i: (0, 0)),
        ],
        out_specs=[
            pl.BlockSpec((None, cout, lout), lambda i: (i, 0, 0)),
            pl.BlockSpec((None, cout, 1), lambda i: (i, 0, 0)),
            pl.BlockSpec((None, cout, 1), lambda i: (i, 0, 0)),
        ],
        out_shape=[
            jax.ShapeDtypeStruct((n, cout, lout), jnp.bfloat16),
            jax.ShapeDtypeStruct((n, cout, 1), jnp.float32),
            jax.ShapeDtypeStruct((n, cout, 1), jnp.float32),
        ],
        compiler_params=pltpu.CompilerParams(
            dimension_semantics=("parallel",), vmem_limit_bytes=vmem_limit),
        cost_estimate=pl.CostEstimate(
            flops=n * (2 * cin * hw * lpad + 2 * 9 * cout * cin * lout
                       + 4 * cout * lout),
            transcendentals=0,
            bytes_accessed=4 * n * cin * hw + 2 * n * cout * lout
                           + 8 * n * cout + 2 * hw * lpad
                           + 2 * 9 * cout * cin + 4 * lout),
    )(x_flat, r_up, w_taps, mask)

    cnt = float(n * ho * wo)
    gamma = bn_gamma.astype(jnp.float32)
    beta = bn_beta.astype(jnp.float32)
    mean = jnp.sum(ch_sum[:, :, 0], axis=0) / cnt
    var = jnp.maximum(jnp.sum(ch_ssq[:, :, 0], axis=0) / cnt - mean * mean, 0.0)
    inv = lax.rsqrt(var + _BN_EPS)
    scale = (gamma * inv).reshape(cout, 1)
    shift = (beta - mean * gamma * inv).reshape(cout, 1)

    y4 = y_conv.reshape(n, cout, ho, wp)
    kern_b = functools.partial(_bn_relu_slice_kernel, wo=wo)
    out = pl.pallas_call(
        kern_b,
        grid=(n,),
        in_specs=[
            pl.BlockSpec((None, cout, ho, wp), lambda i: (i, 0, 0, 0)),
            pl.BlockSpec((cout, 1), lambda i: (0, 0)),
            pl.BlockSpec((cout, 1), lambda i: (0, 0)),
        ],
        out_specs=pl.BlockSpec((None, cout, ho, wo), lambda i: (i, 0, 0, 0)),
        out_shape=jax.ShapeDtypeStruct((n, cout, ho, wo), jnp.float32),
        compiler_params=pltpu.CompilerParams(
            dimension_semantics=("parallel",), vmem_limit_bytes=vmem_limit),
        cost_estimate=pl.CostEstimate(
            flops=3 * n * cout * ho * wo,
            transcendentals=0,
            bytes_accessed=2 * n * cout * ho * wp + 4 * n * cout * ho * wo
                           + 8 * cout),
    )(y4, scale, shift)

    return out


def kernel(x, y, conv_w, conv_b, bn_gamma, bn_beta):
    del conv_b
    return _up_conv_impl(x, conv_w, bn_gamma, bn_beta,
                         out_hw=(int(y.shape[2]), int(y.shape[3])))


# speedup vs baseline: 1.3158x; 1.3158x over previous
import functools

import jax
import jax.numpy as jnp
import numpy as np
from jax import lax
from jax.experimental import pallas as pl
from jax.experimental.pallas import tpu as pltpu

_BN_EPS = 1e-5


def _upconv_stats_kernel(x_ref, r_ref, w_ref, mask_ref, y_ref, sum_ref, ssq_ref,
                         *, wp, lout):
    xb = x_ref[...].astype(jnp.bfloat16)
    xr = jnp.dot(xb, r_ref[...], preferred_element_type=jnp.float32)
    xrb = xr.astype(jnp.bfloat16)
    cout = w_ref.shape[1]
    acc = jnp.zeros((cout, lout), jnp.float32)
    for t in range(9):
        s = (t // 3) * wp + (t % 3)
        acc = acc + jnp.dot(w_ref[t], xrb[:, s:s + lout],
                            preferred_element_type=jnp.float32)
    y_ref[...] = acc.astype(jnp.bfloat16)
    yv = acc * mask_ref[...]
    sum_ref[...] = jnp.sum(yv, axis=1, keepdims=True)
    ssq_ref[...] = jnp.sum(yv * yv, axis=1, keepdims=True)


def _bn_relu_slice_kernel(y_ref, a_ref, b_ref, o_ref, *, wo):
    cout = y_ref.shape[0]
    y = y_ref[:, :, :wo].astype(jnp.float32)
    a = jnp.reshape(a_ref[...], (cout, 1, 1))
    b = jnp.reshape(b_ref[...], (cout, 1, 1))
    o_ref[...] = jnp.maximum(y * a + b, 0.0)


def _interp_matrix_np(in_size, out_size):
    if out_size == 1:
        src = np.zeros((1,), np.float32)
    else:
        src = (np.arange(out_size, dtype=np.float32)
               * np.float32((in_size - 1) / (out_size - 1)))
    i0 = np.clip(np.floor(src).astype(np.int32), 0, in_size - 1)
    i1 = np.clip(i0 + 1, 0, in_size - 1)
    frac = (src - i0.astype(np.float32)).astype(np.float32)
    rows = np.arange(out_size)
    m = np.zeros((out_size, in_size), np.float32)
    np.add.at(m, (rows, i0), np.float32(1.0) - frac)
    np.add.at(m, (rows, i1), frac)
    return m


@functools.lru_cache(maxsize=None)
def _resize_const(h, w, ho, wo):
    wp = wo + 2
    hpp = ho + 3
    lout = ho * wp
    rh = np.zeros((hpp, h), np.float32)
    rh[1:ho + 1] = _interp_matrix_np(h, ho)
    rw = np.zeros((w, wp), np.float32)
    rw[:, 1:wo + 1] = _interp_matrix_np(w, wo).T
    r_up = np.einsum('ih,wj->hwij', rh, rw).reshape(h * w, hpp * wp)
    mask = ((np.arange(lout) % wp) < wo).astype(np.float32).reshape(1, lout)
    return (jnp.asarray(r_up.astype(jnp.bfloat16)), jnp.asarray(mask))


@functools.partial(jax.jit, static_argnames=("out_hw",))
def _up_conv_impl(x, conv_w, bn_gamma, bn_beta, *, out_hw):
    n, cin, h, w = x.shape
    ho, wo = out_hw
    cout = conv_w.shape[0]
    hw = h * w
    wp = wo + 2
    hpp = ho + 3
    lpad = hpp * wp
    lout = ho * wp

    r_up, mask = _resize_const(h, w, ho, wo)

    w_taps = conv_w.astype(jnp.bfloat16).transpose(2, 3, 0, 1).reshape(9, cout, cin)
    x_flat = x.reshape(n, cin, hw)

    vmem_limit = 64 * 1024 * 1024

    kern_a = functools.partial(_upconv_stats_kernel, wp=wp, lout=lout)
    y_conv, ch_sum, ch_ssq = pl.pallas_call(
        kern_a,
        grid=(n,),
        in_specs=[
            pl.BlockSpec((None, cin, hw), lambda i: (i, 0, 0)),
            pl.BlockSpec((hw, lpad), lambda i: (0, 0)),
            pl.BlockSpec((9, cout, cin), lambda i: (0, 0, 0)),
            pl.BlockSpec((1, lout), lambda i: (0, 0)),
        ],
        out_specs=[
            pl.BlockSpec((None, cout, lout), lambda i: (i, 0, 0)),
            pl.BlockSpec((None, cout, 1), lambda i: (i, 0, 0)),
            pl.BlockSpec((None, cout, 1), lambda i: (i, 0, 0)),
        ],
        out_shape=[
            jax.ShapeDtypeStruct((n, cout, lout), jnp.bfloat16),
            jax.ShapeDtypeStruct((n, cout, 1), jnp.float32),
            jax.ShapeDtypeStruct((n, cout, 1), jnp.float32),
        ],
        compiler_params=pltpu.CompilerParams(
            dimension_semantics=("parallel",), vmem_limit_bytes=vmem_limit),
        cost_estimate=pl.CostEstimate(
            flops=n * (2 * cin * hw * lpad + 2 * 9 * cout * cin * lout
                       + 4 * cout * lout),
            transcendentals=0,
            bytes_accessed=4 * n * cin * hw + 2 * n * cout * lout
                           + 8 * n * cout + 2 * hw * lpad
                           + 2 * 9 * cout * cin + 4 * lout),
    )(x_flat, r_up, w_taps, mask)

    cnt = float(n * ho * wo)
    gamma = bn_gamma.astype(jnp.float32)
    beta = bn_beta.astype(jnp.float32)
    mean = jnp.sum(ch_sum[:, :, 0], axis=0) / cnt
    var = jnp.maximum(jnp.sum(ch_ssq[:, :, 0], axis=0) / cnt - mean * mean, 0.0)
    inv = lax.rsqrt(var + _BN_EPS)
    scale = (gamma * inv).reshape(cout, 1)
    shift = (beta - mean * gamma * inv).reshape(cout, 1)

    y4 = y_conv.reshape(n, cout, ho, wp)
    kern_b = functools.partial(_bn_relu_slice_kernel, wo=wo)
    out = pl.pallas_call(
        kern_b,
        grid=(n,),
        in_specs=[
            pl.BlockSpec((None, cout, ho, wp), lambda i: (i, 0, 0, 0)),
            pl.BlockSpec((cout, 1), lambda i: (0, 0)),
            pl.BlockSpec((cout, 1), lambda i: (0, 0)),
        ],
        out_specs=pl.BlockSpec((None, cout, ho, wo), lambda i: (i, 0, 0, 0)),
        out_shape=jax.ShapeDtypeStruct((n, cout, ho, wo), jnp.float32),
        compiler_params=pltpu.CompilerParams(
            dimension_semantics=("parallel",), vmem_limit_bytes=vmem_limit),
        cost_estimate=pl.CostEstimate(
            flops=3 * n * cout * ho * wo,
            transcendentals=0,
            bytes_accessed=2 * n * cout * ho * wp + 4 * n * cout * ho * wo
                           + 8 * cout),
    )(y4, scale, shift)

    return out


def kernel(x, y, conv_w, conv_b, bn_gamma, bn_beta):
    del conv_b
    return _up_conv_impl(x, conv_w, bn_gamma, bn_beta,
                         out_hw=(int(y.shape[2]), int(y.shape[3])))


# speedup vs baseline: 2.7879x; 2.1187x over previous
import functools

import jax
import jax.numpy as jnp
import numpy as np
from jax import lax
from jax.experimental import pallas as pl
from jax.experimental.pallas import tpu as pltpu

_BN_EPS = 1e-5


def _upconv_stats_kernel(x_ref, r_ref, w_ref, y_ref, sum_ref, ssq_ref,
                         *, wp, wo, ho):
    lout = ho * wp
    cout = w_ref.shape[1]
    xb = x_ref[...].astype(jnp.bfloat16)
    xr = lax.dot_general(xb, r_ref[...], (((0,), (0,)), ((), ())),
                         preferred_element_type=jnp.float32)
    xrb = xr.astype(jnp.bfloat16)
    acc = jnp.zeros((cout, lout), jnp.float32)
    for t in range(9):
        s = (t // 3) * wp + (t % 3)
        acc = acc + jnp.dot(w_ref[t], xrb[:, s:s + lout],
                            preferred_element_type=jnp.float32)
    accb = acc.astype(jnp.bfloat16)
    eye = jnp.eye(cout, dtype=jnp.bfloat16)
    acct = lax.dot_general(accb, eye, (((0,), (0,)), ((), ())),
                           preferred_element_type=jnp.float32)
    accc = jnp.concatenate([acct[k * wp:k * wp + wo, :] for k in range(ho)],
                           axis=0)
    y_ref[...] = accc.astype(jnp.bfloat16)
    sum_ref[...] = jnp.sum(accc, axis=0, keepdims=True)
    ssq_ref[...] = jnp.sum(accc * accc, axis=0, keepdims=True)


def _bn_relu_kernel(y_ref, a_ref, b_ref, o_ref):
    yv = y_ref[...].astype(jnp.float32)
    o_ref[...] = jnp.maximum(yv * a_ref[...] + b_ref[...], 0.0)


def _interp_matrix_np(in_size, out_size):
    if out_size == 1:
        src = np.zeros((1,), np.float32)
    else:
        src = (np.arange(out_size, dtype=np.float32)
               * np.float32((in_size - 1) / (out_size - 1)))
    i0 = np.clip(np.floor(src).astype(np.int32), 0, in_size - 1)
    i1 = np.clip(i0 + 1, 0, in_size - 1)
    frac = (src - i0.astype(np.float32)).astype(np.float32)
    rows = np.arange(out_size)
    m = np.zeros((out_size, in_size), np.float32)
    np.add.at(m, (rows, i0), np.float32(1.0) - frac)
    np.add.at(m, (rows, i1), frac)
    return m


@functools.lru_cache(maxsize=None)
def _resize_const(h, w, ho, wo):
    wp = wo + 2
    hpp = ho + 3
    rh = np.zeros((hpp, h), np.float32)
    rh[1:ho + 1] = _interp_matrix_np(h, ho)
    rw = np.zeros((w, wp), np.float32)
    rw[:, 1:wo + 1] = _interp_matrix_np(w, wo).T
    r_up = np.einsum('ih,wj->hwij', rh, rw).reshape(h * w, hpp * wp)
    return jnp.asarray(r_up.astype(jnp.bfloat16))


@functools.partial(jax.jit, static_argnames=("out_hw",))
def _up_conv_impl(x, conv_w, bn_gamma, bn_beta, *, out_hw):
    n, cin, h, w = x.shape
    ho, wo = out_hw
    cout = conv_w.shape[0]
    hw = h * w
    wp = wo + 2
    hpp = ho + 3
    lpad = hpp * wp
    lout = ho * wp
    ldense = ho * wo

    r_upt = _resize_const(h, w, ho, wo)

    x_t = x.transpose(0, 2, 3, 1).reshape(n, hw, cin)
    w9 = conv_w.astype(jnp.bfloat16).transpose(2, 3, 0, 1).reshape(9, cout, cin)

    vmem_limit = 64 * 1024 * 1024

    kern_a = functools.partial(_upconv_stats_kernel, wp=wp, wo=wo, ho=ho)
    y_conv, ch_sum, ch_ssq = pl.pallas_call(
        kern_a,
        grid=(n,),
        in_specs=[
            pl.BlockSpec((None, hw, cin), lambda i: (i, 0, 0)),
            pl.BlockSpec((hw, lpad), lambda i: (0, 0)),
            pl.BlockSpec((9, cout, cin), lambda i: (0, 0, 0)),
        ],
        out_specs=[
            pl.BlockSpec((None, ldense, cout), lambda i: (i, 0, 0)),
            pl.BlockSpec((None, 1, cout), lambda i: (i, 0, 0)),
            pl.BlockSpec((None, 1, cout), lambda i: (i, 0, 0)),
        ],
        out_shape=[
            jax.ShapeDtypeStruct((n, ldense, cout), jnp.bfloat16),
            jax.ShapeDtypeStruct((n, 1, cout), jnp.float32),
            jax.ShapeDtypeStruct((n, 1, cout), jnp.float32),
        ],
        compiler_params=pltpu.CompilerParams(
            dimension_semantics=("parallel",), vmem_limit_bytes=vmem_limit),
        cost_estimate=pl.CostEstimate(
            flops=n * (2 * cin * hw * lpad + 2 * 9 * cout * cin * lout
                       + 4 * cout * ldense),
            transcendentals=0,
            bytes_accessed=4 * n * cin * hw + 2 * n * cout * ldense
                           + 8 * n * cout + 2 * hw * lpad
                           + 2 * 9 * cout * cin),
    )(x_t, r_upt, w9)

    cnt = float(n * ho * wo)
    gamma = bn_gamma.astype(jnp.float32).reshape(1, cout)
    beta = bn_beta.astype(jnp.float32).reshape(1, cout)
    mean = jnp.sum(ch_sum[:, 0, :], axis=0, keepdims=True) / cnt
    var = jnp.maximum(jnp.sum(ch_ssq[:, 0, :], axis=0, keepdims=True) / cnt
                      - mean * mean, 0.0)
    inv = lax.rsqrt(var + _BN_EPS)
    scale = gamma * inv
    shift = beta - mean * gamma * inv

    out_nhwc = pl.pallas_call(
        _bn_relu_kernel,
        grid=(n,),
        in_specs=[
            pl.BlockSpec((None, ldense, cout), lambda i: (i, 0, 0)),
            pl.BlockSpec((1, cout), lambda i: (0, 0)),
            pl.BlockSpec((1, cout), lambda i: (0, 0)),
        ],
        out_specs=pl.BlockSpec((None, ldense, cout), lambda i: (i, 0, 0)),
        out_shape=jax.ShapeDtypeStruct((n, ldense, cout), jnp.float32),
        compiler_params=pltpu.CompilerParams(
            dimension_semantics=("parallel",), vmem_limit_bytes=vmem_limit),
        cost_estimate=pl.CostEstimate(
            flops=3 * n * cout * ldense,
            transcendentals=0,
            bytes_accessed=6 * n * cout * ldense + 8 * cout),
    )(y_conv, scale, shift)

    return out_nhwc.reshape(n, ho, wo, cout).transpose(0, 3, 1, 2)


def kernel(x, y, conv_w, conv_b, bn_gamma, bn_beta):
    del conv_b
    return _up_conv_impl(x, conv_w, bn_gamma, bn_beta,
                         out_hw=(int(y.shape[2]), int(y.shape[3])))
